# baseline (speedup 1.0000x reference)
import functools

import jax
import jax.numpy as jnp
from jax import lax
from jax.experimental import pallas as pl
from jax.experimental.pallas import tpu as pltpu
from jax.experimental.pallas import tpu_sc as plsc

N = 10000
H = 32
K = 15
KP = 16
CIN = 128
COUT = 128
KP_EXTENT = 1.2
E = N * H

NC = 2
NS = 16
NW = NC * NS

CH = 128
NCHUNK = E // CH
MAXC = (NCHUNK + NW - 1) // NW

B = 200
BH = B * H
GB = N // B
G = 8
GH = G * H
NG = B // G


def _sc_gather_body(table_hbm, inds_hbm, xn_hbm, idx_v, rows_v, sem):
    wid = lax.axis_index("s") * NC + lax.axis_index("c")

    def body(i, carry):
        c = wid + i * NW

        @pl.when(c < NCHUNK)
        def _():
            off = pl.multiple_of(c * CH, CH)
            pltpu.sync_copy(inds_hbm.at[pl.ds(off, CH)], idx_v)
            pltpu.async_copy(table_hbm.at[idx_v], rows_v, sem).wait()
            pltpu.sync_copy(rows_v, xn_hbm.at[pl.ds(off, CH)])

        return carry

    lax.fori_loop(0, MAXC, body, 0)


def _sc_gather(table, inds):
    mesh = plsc.VectorSubcoreMesh(core_axis_name="c", subcore_axis_name="s")
    fn = pl.kernel(
        _sc_gather_body,
        mesh=mesh,
        out_type=jax.ShapeDtypeStruct((E, CIN), jnp.float32),
        scratch_types=[
            pltpu.VMEM((CH,), jnp.int32),
            pltpu.VMEM((CH, CIN), jnp.float32),
            pltpu.SemaphoreType.DMA,
        ],
    )
    return fn(table, inds)


def _tc_body(t_ref, w_ref, xn_ref, out_ref):
    raw = xn_ref[...]
    wi = lax.bitcast_convert_type(raw[:, 0:64], jnp.int32)
    f_even = lax.bitcast_convert_type(wi << 16, jnp.float32)
    f_odd = lax.bitcast_convert_type(wi & jnp.int32(-65536), jnp.float32)
    feats = jnp.concatenate([f_even, f_odd], axis=1)
    tb = t_ref[...]
    sx = raw[:, 64:65].reshape(B, H, 1)
    sy = raw[:, 65:66].reshape(B, H, 1)
    sz = raw[:, 66:67].reshape(B, H, 1)
    s2 = raw[:, 67:68].reshape(B, H, 1)
    txe = tb[:, None, 0:16]
    tye = tb[:, None, 16:32]
    tze = tb[:, None, 32:48]
    twe = tb[:, None, 48:64]
    sq3 = s2 + twe + sx * txe + sy * tye + sz * tze
    wgt3 = jnp.maximum(
        1.0 - jnp.sqrt(jnp.maximum(sq3, 0.0)) * (1.0 / KP_EXTENT), 0.0)
    wgt2 = wgt3.reshape(BH, KP)
    c_i = lax.broadcasted_iota(jnp.int32, (GH, G * KP), 0)
    r_i = lax.broadcasted_iota(jnp.int32, (GH, G * KP), 1)
    maskf = ((c_i // H) == (r_i // KP)).astype(jnp.float32)
    parts = []
    for g in range(NG):
        wg = wgt2[g * GH:(g + 1) * GH, :]
        bd = jnp.tile(wg, (1, G)) * maskf
        fg = feats[g * GH:(g + 1) * GH, :]
        parts.append(lax.dot_general(
            bd, fg, (((0,), (0,)), ((), ())),
            preferred_element_type=jnp.float32))
    a2 = jnp.concatenate(parts, axis=0)
    a = a2.reshape(B, KP * CIN)
    out_ref[...] = jnp.dot(a, w_ref[...], preferred_element_type=jnp.float32)


def _tc_call(taug, wflat, xn):
    return pl.pallas_call(
        _tc_body,
        grid=(GB,),
        in_specs=[
            pl.BlockSpec((B, 64), lambda i: (i, 0)),
            pl.BlockSpec((KP * CIN, COUT), lambda i: (0, 0)),
            pl.BlockSpec((BH, CIN), lambda i: (i, 0)),
        ],
        out_specs=pl.BlockSpec((B, COUT), lambda i: (i, 0)),
        out_shape=jax.ShapeDtypeStruct((N, COUT), jnp.float32),
    )(taug, wflat, xn)


def _pack_table(x, s_pts):
    xb = x.astype(jnp.bfloat16)
    packedf = lax.bitcast_convert_type(
        xb.reshape(N, 64, 2), jnp.float32)
    s2 = jnp.sum(s_pts * s_pts, axis=1, keepdims=True)
    row = jnp.concatenate(
        [packedf, s_pts, s2, jnp.zeros((N, 60), jnp.float32)], axis=1)
    shadow = jnp.zeros((1, 128), jnp.float32)
    shadow = shadow.at[0, 64:67].set(1e6).at[0, 67].set(3e12)
    return jnp.concatenate([row, shadow], axis=0)


def _make_taug(q_pts, K_points):
    t = q_pts[:, None, :] + K_points[None, :, :]
    pad0 = jnp.zeros((N, 1), jnp.float32)
    padw = jnp.full((N, 1), 1e30, jnp.float32)
    tx = jnp.concatenate([-2.0 * t[..., 0], pad0], axis=1)
    ty = jnp.concatenate([-2.0 * t[..., 1], pad0], axis=1)
    tz = jnp.concatenate([-2.0 * t[..., 2], pad0], axis=1)
    tw = jnp.concatenate([jnp.sum(t * t, axis=2), padw], axis=1)
    return jnp.concatenate([tx, ty, tz, tw], axis=1)


def kernel(q_pts, s_pts, neighb_inds, x, K_points, W):
    table = _pack_table(x, s_pts)
    inds = neighb_inds.astype(jnp.int32).reshape(E)
    xn = _sc_gather(table, inds)
    taug = _make_taug(q_pts, K_points)
    perm = jnp.arange(CIN).reshape(64, 2).T.reshape(CIN)
    wperm = W[:, perm, :]
    wflat = jnp.concatenate(
        [wperm, jnp.zeros((1, CIN, COUT), jnp.float32)],
        axis=0).reshape(KP * CIN, COUT)
    return _tc_call(taug, wflat, xn)


# speedup vs baseline: 1.9172x; 1.9172x over previous
import functools

import jax
import jax.numpy as jnp
from jax import lax
from jax.experimental import pallas as pl
from jax.experimental.pallas import tpu as pltpu
from jax.experimental.pallas import tpu_sc as plsc

N = 10000
H = 32
K = 15
KP = 16
CIN = 128
COUT = 128
KP_EXTENT = 1.2
E = N * H

NC = 2
NS = 16
NW = NC * NS

CH = 128
NCHUNK = E // CH
MAXC = (NCHUNK + NW - 1) // NW

B = 200
BH = B * H
GB = N // B
G = 8
GH = G * H
GKP = G * KP
NG = B // G


def _sc_gather_body(table_hbm, inds_hbm, xn_hbm, idx_v, rows_v, sem):
    wid = lax.axis_index("s") * NC + lax.axis_index("c")

    def body(i, carry):
        c = wid + i * NW

        @pl.when(c < NCHUNK)
        def _():
            off = pl.multiple_of(c * CH, CH)
            pltpu.sync_copy(inds_hbm.at[pl.ds(off, CH)], idx_v)
            pltpu.async_copy(table_hbm.at[idx_v], rows_v, sem).wait()
            pltpu.sync_copy(rows_v, xn_hbm.at[pl.ds(off, CH)])

        return carry

    lax.fori_loop(0, MAXC, body, 0)


def _sc_gather(table, inds):
    mesh = plsc.VectorSubcoreMesh(core_axis_name="c", subcore_axis_name="s")
    fn = pl.kernel(
        _sc_gather_body,
        mesh=mesh,
        out_type=jax.ShapeDtypeStruct((E, CIN), jnp.float32),
        scratch_types=[
            pltpu.VMEM((CH,), jnp.int32),
            pltpu.VMEM((CH, CIN), jnp.float32),
            pltpu.SemaphoreType.DMA,
        ],
    )
    return fn(table, inds)


def _tc_body(mgt_ref, w_ref, xn_ref, out_ref):
    raw = xn_ref[...]
    wi = lax.bitcast_convert_type(raw[:, 0:64], jnp.int32)
    f_even = lax.bitcast_convert_type(wi << 16, jnp.float32)
    f_odd = lax.bitcast_convert_type(wi & jnp.int32(-65536), jnp.float32)
    feats = jnp.concatenate([f_even, f_odd], axis=1)
    saug = raw[:, 64:69]
    mgt = mgt_ref[...]
    c_i = lax.broadcasted_iota(jnp.int32, (GH, GKP), 0)
    r_i = lax.broadcasted_iota(jnp.int32, (GH, GKP), 1)
    maskf = ((c_i // H) == (r_i // KP)).astype(jnp.float32)
    parts = []
    for g in range(NG):
        sg = saug[g * GH:(g + 1) * GH, :]
        mg = mgt[:, g * GKP:(g + 1) * GKP]
        sqb = jnp.dot(sg, mg, preferred_element_type=jnp.float32)
        wb = jnp.maximum(
            1.0 - jnp.sqrt(jnp.maximum(sqb, 0.0)) * (1.0 / KP_EXTENT),
            0.0) * maskf
        fg = feats[g * GH:(g + 1) * GH, :]
        parts.append(lax.dot_general(
            wb, fg, (((0,), (0,)), ((), ())),
            preferred_element_type=jnp.float32))
    a2 = jnp.concatenate(parts, axis=0)
    a = a2.reshape(B, KP * CIN)
    out_ref[...] = jnp.dot(a, w_ref[...], preferred_element_type=jnp.float32)


def _tc_call(mgt, wflat, xn):
    return pl.pallas_call(
        _tc_body,
        grid=(GB,),
        in_specs=[
            pl.BlockSpec((5, B * KP), lambda i: (0, i)),
            pl.BlockSpec((KP * CIN, COUT), lambda i: (0, 0)),
            pl.BlockSpec((BH, CIN), lambda i: (i, 0)),
        ],
        out_specs=pl.BlockSpec((B, COUT), lambda i: (i, 0)),
        out_shape=jax.ShapeDtypeStruct((N, COUT), jnp.float32),
    )(mgt, wflat, xn)


def _pack_table(x, s_pts):
    xb = x.astype(jnp.bfloat16)
    packedf = lax.bitcast_convert_type(
        xb.reshape(N, 64, 2), jnp.float32)
    s2 = jnp.sum(s_pts * s_pts, axis=1, keepdims=True)
    ones = jnp.ones((N, 1), jnp.float32)
    row = jnp.concatenate(
        [packedf, s_pts, s2, ones, jnp.zeros((N, 59), jnp.float32)], axis=1)
    shadow = jnp.zeros((1, 128), jnp.float32)
    shadow = shadow.at[0, 64:67].set(1e6).at[0, 67].set(3e12)
    shadow = shadow.at[0, 68].set(1.0)
    return jnp.concatenate([row, shadow], axis=0)


def _make_maugt(q_pts, K_points):
    t = q_pts[:, None, :] + K_points[None, :, :]
    padc = jnp.zeros((N, 1), jnp.float32)
    padw = jnp.full((N, 1), 1e30, jnp.float32)
    tx = jnp.concatenate([-2.0 * t[..., 0], padc], axis=1)
    ty = jnp.concatenate([-2.0 * t[..., 1], padc], axis=1)
    tz = jnp.concatenate([-2.0 * t[..., 2], padc], axis=1)
    tw = jnp.concatenate([jnp.sum(t * t, axis=2), padw], axis=1)
    on = jnp.ones((N, KP), jnp.float32)
    return jnp.stack(
        [tx.reshape(-1), ty.reshape(-1), tz.reshape(-1),
         on.reshape(-1), tw.reshape(-1)], axis=0)


def kernel(q_pts, s_pts, neighb_inds, x, K_points, W):
    table = _pack_table(x, s_pts)
    inds = neighb_inds.astype(jnp.int32).reshape(E)
    xn = _sc_gather(table, inds)
    mgt = _make_maugt(q_pts, K_points)
    perm = jnp.arange(CIN).reshape(64, 2).T.reshape(CIN)
    wperm = W[:, perm, :]
    wflat = jnp.concatenate(
        [wperm, jnp.zeros((1, CIN, COUT), jnp.float32)],
        axis=0).reshape(KP * CIN, COUT)
    return _tc_call(mgt, wflat, xn)
